# baseline (speedup 1.0000x reference)
import jax
import jax.numpy as jnp
from jax import lax
from jax.experimental import pallas as pl
from jax.experimental.pallas import tpu as pltpu
from jax.experimental.pallas import tpu_sc as plsc

_N = 10000
_E = 320000
_NC = 2
_NS = 16
_NW = _NC * _NS
_RPT = 320
_NPAD = _NW * _RPT
_RPC = _NS * _RPT
_B = 128
_NSLOT = 4
_NF = 2


def _lower_bound(rows_hbm, bs_v, target):
    def step(_, carry):
        lo, hi = carry
        mid = jnp.minimum((lo + hi) // 2, _E - 1)
        base = (mid // 16) * 16
        pltpu.sync_copy(rows_hbm.at[pl.ds(base, 16)], bs_v.at[pl.ds(0, 16)])
        rv = bs_v[pl.ds(mid - base, 16)][0]
        valid = lo < hi
        less = rv < target
        lo = jnp.where(valid & less, mid + 1, lo)
        hi = jnp.where(valid & jnp.logical_not(less), mid, hi)
        return lo, hi

    lo, _ = lax.fori_loop(0, 19, step, (jnp.int32(0), jnp.int32(_E)))
    return lo


def _make_spmm(d, with_bias, emit_offs):
    mesh = plsc.VectorSubcoreMesh(
        core_axis_name="c", subcore_axis_name="s", num_cores=_NC, num_subcores=_NS
    )

    def body(*refs):
        it = iter(refs)
        h_hbm = next(it)
        rows_hbm = next(it)
        cols_hbm = next(it)
        vals_hbm = next(it)
        offs_in = None if emit_offs else next(it)
        bias_hbm = next(it) if with_bias else None
        out_hbm = next(it)
        offs_out = next(it) if emit_offs else None
        bs_v = next(it)
        bias_v = next(it) if with_bias else None
        cols = [next(it) for _ in range(_NSLOT)]
        rows_s = [next(it) for _ in range(_NSLOT)]
        lr = [next(it) for _ in range(_NSLOT)]
        vals = [next(it) for _ in range(_NSLOT)]
        gb = [next(it) for _ in range(_NSLOT)]
        gf = [next(it) for _ in range(_NF)]
        acc_sh = next(it)
        sem_i = [next(it) for _ in range(_NSLOT)]
        sem_g = [next(it) for _ in range(_NSLOT)]
        sem_s = [next(it) for _ in range(_NF)]

        c = lax.axis_index("c")
        s = lax.axis_index("s")
        wid = c * _NS + s
        row_lo = wid * _RPT

        if with_bias:
            pltpu.sync_copy(bias_hbm, bias_v)
            ivecs = [bias_v[pl.ds(j * 16, 16)] for j in range(d // 16)]
        else:
            ivecs = [jnp.zeros((16,), jnp.float32)] * (d // 16)

        def init_row(r, carry):
            for j in range(d // 16):
                gf[0][r, pl.ds(j * 16, 16)] = ivecs[j]
            return carry

        lax.fori_loop(0, 64, init_row, 0)
        for k in range(_RPT // 64):
            pltpu.sync_copy(
                gf[0].at[pl.ds(0, 64)], acc_sh.at[pl.ds(s * _RPT + k * 64, 64)]
            )

        if emit_offs:
            e_start = _lower_bound(rows_hbm, bs_v, row_lo)
            e_end = _lower_bound(rows_hbm, bs_v, row_lo + _RPT)
            vec = jnp.where(lax.iota(jnp.int32, 16) == 0, e_start, e_end)
            bs_v[pl.ds(0, 16)] = vec
            pltpu.sync_copy(bs_v.at[pl.ds(0, 16)], offs_out.at[pl.ds(wid * 16, 16)])
        else:
            pltpu.sync_copy(offs_in.at[pl.ds(wid * 16, 16)], bs_v.at[pl.ds(0, 16)])
            e_start = bs_v[pl.ds(0, 16)][0]
            e_end = bs_v[pl.ds(1, 16)][0]

        es_al = (e_start // 8) * 8
        nb = (e_end - es_al + _B - 1) // _B

        def e0_of(b):
            return jnp.minimum(es_al + b * _B, _E - _B)

        def idx_start(b, j):
            e0 = e0_of(b)
            pltpu.async_copy(cols_hbm.at[pl.ds(e0, _B)], cols[j], sem_i[j])
            pltpu.async_copy(rows_hbm.at[pl.ds(e0, _B)], rows_s[j], sem_i[j])
            pltpu.async_copy(
                vals_hbm.at[pl.ds(e0, _B)], vals[j].at[pl.ds(0, _B)], sem_i[j]
            )

        def idx_wait(j):
            pltpu.make_async_copy(cols_hbm.at[pl.ds(0, _B)], cols[j], sem_i[j]).wait()
            pltpu.make_async_copy(rows_hbm.at[pl.ds(0, _B)], rows_s[j], sem_i[j]).wait()
            pltpu.make_async_copy(
                vals_hbm.at[pl.ds(0, _B)], vals[j].at[pl.ds(0, _B)], sem_i[j]
            ).wait()

        def mask(b, j):
            e0 = e0_of(b)
            e0_nom = es_al + b * _B
            for gi in range(_B // 16):
                lane_e = e0 + gi * 16 + lax.iota(jnp.int32, 16)
                valid = (lane_e >= e_start) & (lane_e < e_end) & (lane_e >= e0_nom)
                sl = pl.ds(gi * 16, 16)
                cols[j][sl] = jnp.where(valid, cols[j][sl], 0)
                vals[j][sl] = jnp.where(valid, vals[j][sl], 0.0)
                lrv = jnp.clip(rows_s[j][sl] - row_lo, 0, _RPT - 1) + s * _RPT
                lr[j][0, sl] = lrv

        def gather_start(j):
            hf = _B // 2
            pltpu.async_copy(
                h_hbm.at[cols[j].at[pl.ds(0, hf)]], gb[j].at[pl.ds(0, hf)],
                sem_g[j],
            )
            pltpu.async_copy(
                h_hbm.at[cols[j].at[pl.ds(hf, hf)]], gb[j].at[pl.ds(hf, hf)],
                sem_g[j],
            )

        def gather_wait(j):
            pltpu.make_async_copy(h_hbm.at[pl.ds(0, _B)], gb[j], sem_g[j]).wait()

        def scale(j, k):
            hi_mask = jnp.full((16,), -65536, jnp.int32)
            it16 = lax.iota(jnp.int32, 16)

            @plsc.parallel_loop(0, _B, unroll=4)
            def _(e):
                v = vals[j][pl.ds(e, 16)][0]
                erow = jnp.full((16,), e, jnp.int32)
                for gi in range(d // 32):
                    v32 = gb[j][e, pl.ds(gi * 32, 32)]
                    vi = plsc.bitcast(v32, jnp.int32)
                    even = plsc.bitcast(vi << 16, jnp.float32)
                    odd = plsc.bitcast(vi & hi_mask, jnp.float32)
                    plsc.store_scatter(
                        gf[k], [erow, gi * 32 + 2 * it16], even * v
                    )
                    plsc.store_scatter(
                        gf[k], [erow, gi * 32 + 1 + 2 * it16], odd * v
                    )

        def scatter_start(j, k):
            pltpu.async_copy(gf[k], acc_sh.at[lr[j].at[0]], sem_s[k], add=True)

        def scatter_wait(k):
            pltpu.make_async_copy(h_hbm.at[pl.ds(0, _B)], gf[k], sem_s[k]).wait()

        for p in range(_NSLOT - 1):
            @pl.when(nb > p)
            def _(p=p):
                idx_start(p, p)

        @pl.when(nb > 0)
        def _():
            idx_wait(0)
            mask(0, 0)
            gather_start(0)

        def outer(i, carry):
            for jj in range(_NSLOT):
                b = _NSLOT * i + jj
                j = jj
                j1 = (jj + 1) % _NSLOT
                k = jj % _NF

                @pl.when(b < nb)
                def _():
                    gather_wait(j)

                    @pl.when(b + 1 < nb)
                    def _():
                        idx_wait(j1)
                        mask(b + 1, j1)
                        gather_start(j1)

                    @pl.when(b >= _NF)
                    def _():
                        scatter_wait(k)

                    scale(j, k)
                    scatter_start(j, k)

                    @pl.when(b + _NSLOT - 1 < nb)
                    def _():
                        idx_start(b + _NSLOT - 1, (jj + _NSLOT - 1) % _NSLOT)

            return carry

        lax.fori_loop(0, (nb + _NSLOT - 1) // _NSLOT, outer, 0)

        for kk in range(_NF):
            @pl.when(nb > kk)
            def _(kk=kk):
                scatter_wait(kk)

        pltpu.sync_copy(
            acc_sh.at[pl.ds(s * _RPT, _RPT)], out_hbm.at[pl.ds(row_lo, _RPT)]
        )

    scratch = [
        pltpu.VMEM((32,), jnp.int32),
    ]
    if with_bias:
        scratch.append(pltpu.VMEM((d,), jnp.float32))
    scratch += [pltpu.VMEM((_B,), jnp.int32) for _ in range(_NSLOT)]
    scratch += [pltpu.VMEM((_B,), jnp.int32) for _ in range(_NSLOT)]
    scratch += [pltpu.VMEM((1, _B), jnp.int32) for _ in range(_NSLOT)]
    scratch += [pltpu.VMEM((_B + 16,), jnp.float32) for _ in range(_NSLOT)]
    scratch += [pltpu.VMEM((_B, d), jnp.bfloat16) for _ in range(_NSLOT)]
    scratch += [pltpu.VMEM((_B, d), jnp.float32) for _ in range(_NF)]
    scratch += [pltpu.VMEM_SHARED((_RPC, d), jnp.float32)]
    scratch += [pltpu.SemaphoreType.DMA for _ in range(2 * _NSLOT + _NF)]

    out_type = jax.ShapeDtypeStruct((_NPAD, d), jnp.float32)
    if emit_offs:
        out_type = [out_type, jax.ShapeDtypeStruct((_NW * 16,), jnp.int32)]

    return pl.kernel(
        body,
        out_type=out_type,
        mesh=mesh,
        scratch_types=scratch,
        compiler_params=pltpu.CompilerParams(
            use_tc_tiling_on_sc=False, needs_layout_passes=False
        ),
    )


def _tc_dense(h_ref, w1t_ref, b1_ref, w2t_ref, o_ref):
    h = h_ref[...]
    z = jnp.dot(h, w1t_ref[...], preferred_element_type=jnp.float32)
    z = jnp.maximum(z + b1_ref[...], 0.0)
    o_ref[...] = jnp.dot(z, w2t_ref[...], preferred_element_type=jnp.float32)


@jax.jit
def kernel(x, propagation_adj, filter_vals, W1, b1, W2, b2, filter_rows, filter_cols):
    del propagation_adj
    d_hid = W1.shape[0]
    n_cls = W2.shape[0]

    spmm1 = _make_spmm(d_hid, with_bias=False, emit_offs=True)
    spmm2 = _make_spmm(n_cls, with_bias=True, emit_offs=False)

    x_b = x.astype(jnp.bfloat16)
    h1, offs = spmm1(x_b, filter_rows, filter_cols, filter_vals)

    t = pl.pallas_call(
        _tc_dense,
        out_shape=jax.ShapeDtypeStruct((_NPAD, n_cls), jnp.float32),
    )(h1, W1.T, b1[None, :], W2.T)

    outp = spmm2(
        t.astype(jnp.bfloat16), filter_rows, filter_cols, filter_vals, offs, b2
    )
    return outp[:_N]


# speedup vs baseline: 1.0015x; 1.0015x over previous
import jax
import jax.numpy as jnp
from jax import lax
from jax.experimental import pallas as pl
from jax.experimental.pallas import tpu as pltpu
from jax.experimental.pallas import tpu_sc as plsc

_N = 10000
_E = 320000
_NC = 2
_NS = 16
_NW = _NC * _NS
_RPT = 320
_NPAD = _NW * _RPT
_RPC = _NS * _RPT
_B = 128
_NSLOT = 4
_NF = 2


def _lower_bound(rows_hbm, bs_v, target):
    def step(_, carry):
        lo, hi = carry
        mid = jnp.minimum((lo + hi) // 2, _E - 1)
        base = (mid // 16) * 16
        pltpu.sync_copy(rows_hbm.at[pl.ds(base, 16)], bs_v.at[pl.ds(0, 16)])
        rv = bs_v[pl.ds(mid - base, 16)][0]
        valid = lo < hi
        less = rv < target
        lo = jnp.where(valid & less, mid + 1, lo)
        hi = jnp.where(valid & jnp.logical_not(less), mid, hi)
        return lo, hi

    lo, _ = lax.fori_loop(0, 19, step, (jnp.int32(0), jnp.int32(_E)))
    return lo


def _make_spmm(d, with_bias, emit_offs):
    mesh = plsc.VectorSubcoreMesh(
        core_axis_name="c", subcore_axis_name="s", num_cores=_NC, num_subcores=_NS
    )

    def body(*refs):
        it = iter(refs)
        h_hbm = next(it)
        rows_hbm = next(it)
        cols_hbm = next(it)
        vals_hbm = next(it)
        offs_in = None if emit_offs else next(it)
        bias_hbm = next(it) if with_bias else None
        out_hbm = next(it)
        offs_out = next(it) if emit_offs else None
        bs_v = next(it)
        bias_v = next(it) if with_bias else None
        cols = [next(it) for _ in range(_NSLOT)]
        rows_s = [next(it) for _ in range(_NSLOT)]
        lr = [next(it) for _ in range(_NSLOT)]
        vals = [next(it) for _ in range(_NSLOT)]
        gb = [next(it) for _ in range(_NSLOT)]
        gf = [next(it) for _ in range(_NF)]
        acc_sh = next(it)
        sem_i = [next(it) for _ in range(_NSLOT)]
        sem_g = [next(it) for _ in range(_NSLOT)]
        sem_s = [next(it) for _ in range(_NF)]

        c = lax.axis_index("c")
        s = lax.axis_index("s")
        wid = c * _NS + s
        row_lo = wid * _RPT

        if with_bias:
            pltpu.sync_copy(bias_hbm, bias_v)
            ivecs = [bias_v[pl.ds(j * 16, 16)] for j in range(d // 16)]
        else:
            ivecs = [jnp.zeros((16,), jnp.float32)] * (d // 16)

        def init_row(r, carry):
            for j in range(d // 16):
                gf[0][r, pl.ds(j * 16, 16)] = ivecs[j]
            return carry

        lax.fori_loop(0, 64, init_row, 0)
        for k in range(_RPT // 64):
            pltpu.sync_copy(
                gf[0].at[pl.ds(0, 64)], acc_sh.at[pl.ds(s * _RPT + k * 64, 64)]
            )

        if emit_offs:
            e_start = _lower_bound(rows_hbm, bs_v, row_lo)
            e_end = _lower_bound(rows_hbm, bs_v, row_lo + _RPT)
            vec = jnp.where(lax.iota(jnp.int32, 16) == 0, e_start, e_end)
            bs_v[pl.ds(0, 16)] = vec
            pltpu.sync_copy(bs_v.at[pl.ds(0, 16)], offs_out.at[pl.ds(wid * 16, 16)])
        else:
            pltpu.sync_copy(offs_in.at[pl.ds(wid * 16, 16)], bs_v.at[pl.ds(0, 16)])
            e_start = bs_v[pl.ds(0, 16)][0]
            e_end = bs_v[pl.ds(1, 16)][0]

        es_al = (e_start // 8) * 8
        nb = (e_end - es_al + _B - 1) // _B

        def e0_of(b):
            return jnp.minimum(es_al + b * _B, _E - _B)

        def idx_start(b, j):
            e0 = e0_of(b)
            pltpu.async_copy(cols_hbm.at[pl.ds(e0, _B)], cols[j], sem_i[j])
            pltpu.async_copy(rows_hbm.at[pl.ds(e0, _B)], rows_s[j], sem_i[j])
            pltpu.async_copy(
                vals_hbm.at[pl.ds(e0, _B)], vals[j].at[pl.ds(0, _B)], sem_i[j]
            )

        def idx_wait(j):
            pltpu.make_async_copy(cols_hbm.at[pl.ds(0, _B)], cols[j], sem_i[j]).wait()
            pltpu.make_async_copy(rows_hbm.at[pl.ds(0, _B)], rows_s[j], sem_i[j]).wait()
            pltpu.make_async_copy(
                vals_hbm.at[pl.ds(0, _B)], vals[j].at[pl.ds(0, _B)], sem_i[j]
            ).wait()

        def mask(b, j):
            e0 = e0_of(b)
            e0_nom = es_al + b * _B
            for gi in range(_B // 16):
                lane_e = e0 + gi * 16 + lax.iota(jnp.int32, 16)
                valid = (lane_e >= e_start) & (lane_e < e_end) & (lane_e >= e0_nom)
                sl = pl.ds(gi * 16, 16)
                cols[j][sl] = jnp.where(valid, cols[j][sl], 0)
                vals[j][sl] = jnp.where(valid, vals[j][sl], 0.0)
                lrv = jnp.clip(rows_s[j][sl] - row_lo, 0, _RPT - 1) + s * _RPT
                lr[j][0, sl] = lrv

        def gather_start(j):
            pltpu.async_copy(h_hbm.at[cols[j]], gb[j], sem_g[j])

        def gather_wait(j):
            pltpu.make_async_copy(h_hbm.at[pl.ds(0, _B)], gb[j], sem_g[j]).wait()

        def scale(j, k):
            hi_mask = jnp.full((16,), -65536, jnp.int32)
            it16 = lax.iota(jnp.int32, 16)

            @plsc.parallel_loop(0, _B, unroll=4)
            def _(e):
                v = vals[j][pl.ds(e, 16)][0]
                erow = jnp.full((16,), e, jnp.int32)
                for gi in range(d // 32):
                    v32 = gb[j][e, pl.ds(gi * 32, 32)]
                    vi = plsc.bitcast(v32, jnp.int32)
                    even = plsc.bitcast(vi << 16, jnp.float32)
                    odd = plsc.bitcast(vi & hi_mask, jnp.float32)
                    plsc.store_scatter(
                        gf[k], [erow, gi * 32 + 2 * it16], even * v
                    )
                    plsc.store_scatter(
                        gf[k], [erow, gi * 32 + 1 + 2 * it16], odd * v
                    )

        def scatter_start(j, k):
            pltpu.async_copy(gf[k], acc_sh.at[lr[j].at[0]], sem_s[k], add=True)

        def scatter_wait(k):
            pltpu.make_async_copy(h_hbm.at[pl.ds(0, _B)], gf[k], sem_s[k]).wait()

        for p in range(_NSLOT - 1):
            @pl.when(nb > p)
            def _(p=p):
                idx_start(p, p)

        @pl.when(nb > 0)
        def _():
            idx_wait(0)
            mask(0, 0)
            gather_start(0)

        def outer(i, carry):
            for jj in range(_NSLOT):
                b = _NSLOT * i + jj
                j = jj
                j1 = (jj + 1) % _NSLOT
                k = jj % _NF

                @pl.when(b < nb)
                def _():
                    gather_wait(j)

                    @pl.when(b + 1 < nb)
                    def _():
                        idx_wait(j1)
                        mask(b + 1, j1)
                        gather_start(j1)

                    @pl.when(b >= _NF)
                    def _():
                        scatter_wait(k)

                    scale(j, k)
                    scatter_start(j, k)

                    @pl.when(b + _NSLOT - 1 < nb)
                    def _():
                        idx_start(b + _NSLOT - 1, (jj + _NSLOT - 1) % _NSLOT)

            return carry

        lax.fori_loop(0, (nb + _NSLOT - 1) // _NSLOT, outer, 0)

        for kk in range(_NF):
            @pl.when(nb > kk)
            def _(kk=kk):
                scatter_wait(kk)

        pltpu.sync_copy(
            acc_sh.at[pl.ds(s * _RPT, _RPT)], out_hbm.at[pl.ds(row_lo, _RPT)]
        )

    scratch = [
        pltpu.VMEM((32,), jnp.int32),
    ]
    if with_bias:
        scratch.append(pltpu.VMEM((d,), jnp.float32))
    scratch += [pltpu.VMEM((_B,), jnp.int32) for _ in range(_NSLOT)]
    scratch += [pltpu.VMEM((_B,), jnp.int32) for _ in range(_NSLOT)]
    scratch += [pltpu.VMEM((1, _B), jnp.int32) for _ in range(_NSLOT)]
    scratch += [pltpu.VMEM((_B + 16,), jnp.float32) for _ in range(_NSLOT)]
    scratch += [pltpu.VMEM((_B, d), jnp.bfloat16) for _ in range(_NSLOT)]
    scratch += [pltpu.VMEM((_B, d), jnp.float32) for _ in range(_NF)]
    scratch += [pltpu.VMEM_SHARED((_RPC, d), jnp.float32)]
    scratch += [pltpu.SemaphoreType.DMA for _ in range(2 * _NSLOT + _NF)]

    out_type = jax.ShapeDtypeStruct((_NPAD, d), jnp.float32)
    if emit_offs:
        out_type = [out_type, jax.ShapeDtypeStruct((_NW * 16,), jnp.int32)]

    return pl.kernel(
        body,
        out_type=out_type,
        mesh=mesh,
        scratch_types=scratch,
        compiler_params=pltpu.CompilerParams(
            use_tc_tiling_on_sc=False, needs_layout_passes=False
        ),
    )


def _tc_dense(h_ref, w1t_ref, b1_ref, w2t_ref, o_ref):
    h = h_ref[...]
    z = jnp.dot(h, w1t_ref[...], preferred_element_type=jnp.float32)
    z = jnp.maximum(z + b1_ref[...], 0.0)
    o_ref[...] = jnp.dot(z, w2t_ref[...], preferred_element_type=jnp.float32)


@jax.jit
def kernel(x, propagation_adj, filter_vals, W1, b1, W2, b2, filter_rows, filter_cols):
    del propagation_adj
    d_hid = W1.shape[0]
    n_cls = W2.shape[0]

    spmm1 = _make_spmm(d_hid, with_bias=False, emit_offs=True)
    spmm2 = _make_spmm(n_cls, with_bias=True, emit_offs=False)

    x_b = x.astype(jnp.bfloat16)
    h1, offs = spmm1(x_b, filter_rows, filter_cols, filter_vals)

    t = pl.pallas_call(
        _tc_dense,
        out_shape=jax.ShapeDtypeStruct((_NPAD, n_cls), jnp.float32),
    )(h1, W1.T, b1[None, :], W2.T)

    outp = spmm2(
        t.astype(jnp.bfloat16), filter_rows, filter_cols, filter_vals, offs, b2
    )
    return outp[:_N]


# speedup vs baseline: 1.0057x; 1.0042x over previous
import jax
import jax.numpy as jnp
from jax import lax
from jax.experimental import pallas as pl
from jax.experimental.pallas import tpu as pltpu
from jax.experimental.pallas import tpu_sc as plsc

_N = 10000
_E = 320000
_NC = 2
_NS = 16
_NW = _NC * _NS
_RPT = 320
_NPAD = _NW * _RPT
_RPC = _NS * _RPT
_B = 128
_NSLOT = 4
_NF = 2


def _lower_bound(rows_hbm, bs_v, target):
    def step(_, carry):
        lo, hi = carry
        mid = jnp.minimum((lo + hi) // 2, _E - 1)
        base = (mid // 16) * 16
        pltpu.sync_copy(rows_hbm.at[pl.ds(base, 16)], bs_v.at[pl.ds(0, 16)])
        rv = bs_v[pl.ds(mid - base, 16)][0]
        valid = lo < hi
        less = rv < target
        lo = jnp.where(valid & less, mid + 1, lo)
        hi = jnp.where(valid & jnp.logical_not(less), mid, hi)
        return lo, hi

    lo, _ = lax.fori_loop(0, 19, step, (jnp.int32(0), jnp.int32(_E)))
    return lo


def _make_spmm(d, with_bias, emit_offs):
    mesh = plsc.VectorSubcoreMesh(
        core_axis_name="c", subcore_axis_name="s", num_cores=_NC, num_subcores=_NS
    )

    def body(*refs):
        it = iter(refs)
        h_hbm = next(it)
        rows_hbm = next(it)
        cols_hbm = next(it)
        vals_hbm = next(it)
        offs_in = None if emit_offs else next(it)
        bias_hbm = next(it) if with_bias else None
        out_hbm = next(it)
        offs_out = next(it) if emit_offs else None
        bs_v = next(it)
        bias_v = next(it) if with_bias else None
        cols = [next(it) for _ in range(_NSLOT)]
        rows_s = [next(it) for _ in range(_NSLOT)]
        lr = [next(it) for _ in range(_NSLOT)]
        vals = [next(it) for _ in range(_NSLOT)]
        gb = [next(it) for _ in range(_NSLOT)]
        gf = [next(it) for _ in range(_NF)]
        acc_sh = next(it)
        sem_i = [next(it) for _ in range(_NSLOT)]
        sem_g = [next(it) for _ in range(_NSLOT)]
        sem_s = [next(it) for _ in range(_NF)]

        c = lax.axis_index("c")
        s = lax.axis_index("s")
        wid = c * _NS + s
        row_lo = wid * _RPT

        if with_bias:
            pltpu.sync_copy(bias_hbm, bias_v)
            ivecs = [bias_v[pl.ds(j * 16, 16)] for j in range(d // 16)]
        else:
            ivecs = [jnp.zeros((16,), jnp.float32)] * (d // 16)

        def init_row(r, carry):
            for j in range(d // 16):
                gf[0][r, pl.ds(j * 16, 16)] = ivecs[j]
            return carry

        lax.fori_loop(0, 64, init_row, 0)
        for k in range(_RPT // 64):
            pltpu.sync_copy(
                gf[0].at[pl.ds(0, 64)], acc_sh.at[pl.ds(s * _RPT + k * 64, 64)]
            )

        if emit_offs:
            e_start = _lower_bound(rows_hbm, bs_v, row_lo)
            e_end = _lower_bound(rows_hbm, bs_v, row_lo + _RPT)
            vec = jnp.where(lax.iota(jnp.int32, 16) == 0, e_start, e_end)
            bs_v[pl.ds(0, 16)] = vec
            pltpu.sync_copy(bs_v.at[pl.ds(0, 16)], offs_out.at[pl.ds(wid * 16, 16)])
        else:
            pltpu.sync_copy(offs_in.at[pl.ds(wid * 16, 16)], bs_v.at[pl.ds(0, 16)])
            e_start = bs_v[pl.ds(0, 16)][0]
            e_end = bs_v[pl.ds(1, 16)][0]

        es_al = (e_start // 8) * 8
        nb = (e_end - es_al + _B - 1) // _B

        def e0_of(b):
            return jnp.minimum(es_al + b * _B, _E - _B)

        def idx_start(b, j):
            e0 = e0_of(b)
            pltpu.async_copy(cols_hbm.at[pl.ds(e0, _B)], cols[j], sem_i[j])
            pltpu.async_copy(rows_hbm.at[pl.ds(e0, _B)], rows_s[j], sem_i[j])
            pltpu.async_copy(
                vals_hbm.at[pl.ds(e0, _B)], vals[j].at[pl.ds(0, _B)], sem_i[j]
            )

        def idx_wait(j):
            pltpu.make_async_copy(cols_hbm.at[pl.ds(0, _B)], cols[j], sem_i[j]).wait()
            pltpu.make_async_copy(rows_hbm.at[pl.ds(0, _B)], rows_s[j], sem_i[j]).wait()
            pltpu.make_async_copy(
                vals_hbm.at[pl.ds(0, _B)], vals[j].at[pl.ds(0, _B)], sem_i[j]
            ).wait()

        def mask(b, j):
            e0 = e0_of(b)
            e0_nom = es_al + b * _B
            for gi in range(_B // 16):
                lane_e = e0 + gi * 16 + lax.iota(jnp.int32, 16)
                valid = (lane_e >= e_start) & (lane_e < e_end) & (lane_e >= e0_nom)
                sl = pl.ds(gi * 16, 16)
                cols[j][sl] = jnp.where(valid, cols[j][sl], 0)
                vals[j][sl] = jnp.where(valid, vals[j][sl], 0.0)
                lrv = jnp.clip(rows_s[j][sl] - row_lo, 0, _RPT - 1) + s * _RPT
                lr[j][0, sl] = lrv

        def gather_start(j):
            pltpu.async_copy(h_hbm.at[cols[j]], gb[j], sem_g[j])

        def gather_wait(j):
            pltpu.make_async_copy(h_hbm.at[pl.ds(0, _B)], gb[j], sem_g[j]).wait()

        def scale(j, k):
            hi_mask = jnp.full((16,), -65536, jnp.int32)
            it16 = lax.iota(jnp.int32, 16)

            @plsc.parallel_loop(0, _B, unroll=4)
            def _(e):
                v = vals[j][pl.ds(e, 16)][0]
                erow = jnp.full((16,), e, jnp.int32)
                for gi in range(d // 32):
                    v32 = gb[j][e, pl.ds(gi * 32, 32)]
                    vi = plsc.bitcast(v32, jnp.int32)
                    even = plsc.bitcast(vi << 16, jnp.float32)
                    odd = plsc.bitcast(vi & hi_mask, jnp.float32)
                    plsc.store_scatter(
                        gf[k], [erow, gi * 32 + 2 * it16], even * v
                    )
                    plsc.store_scatter(
                        gf[k], [erow, gi * 32 + 1 + 2 * it16], odd * v
                    )

        def scatter_start(j, k):
            pltpu.async_copy(gf[k], acc_sh.at[lr[j].at[0]], sem_s[k], add=True)

        def scatter_wait(k):
            pltpu.make_async_copy(h_hbm.at[pl.ds(0, _B)], gf[k], sem_s[k]).wait()

        for p in range(_NSLOT - 1):
            @pl.when(nb > p)
            def _(p=p):
                idx_start(p, p)

        @pl.when(nb > 0)
        def _():
            idx_wait(0)
            mask(0, 0)
            gather_start(0)

        def outer(i, carry):
            for jj in range(_NSLOT):
                b = _NSLOT * i + jj
                j = jj
                j1 = (jj + 1) % _NSLOT
                k = jj % _NF

                @pl.when(b < nb)
                def _():
                    gather_wait(j)

                    @pl.when(b + 1 < nb)
                    def _():
                        idx_wait(j1)
                        mask(b + 1, j1)
                        gather_start(j1)

                    @pl.when(b >= _NF)
                    def _():
                        scatter_wait(k)

                    scale(j, k)
                    scatter_start(j, k)

                    @pl.when(b + _NSLOT - 1 < nb)
                    def _():
                        idx_start(b + _NSLOT - 1, (jj + _NSLOT - 1) % _NSLOT)

            return carry

        lax.fori_loop(0, (nb + _NSLOT - 1) // _NSLOT, outer, 0)

        for kk in range(_NF):
            @pl.when(nb > kk)
            def _(kk=kk):
                scatter_wait(kk)

        pltpu.sync_copy(
            acc_sh.at[pl.ds(s * _RPT, _RPT)], out_hbm.at[pl.ds(row_lo, _RPT)]
        )

    scratch = [
        pltpu.VMEM((32,), jnp.int32),
    ]
    if with_bias:
        scratch.append(pltpu.VMEM((d,), jnp.float32))
    scratch += [pltpu.VMEM((_B,), jnp.int32) for _ in range(_NSLOT)]
    scratch += [pltpu.VMEM((_B,), jnp.int32) for _ in range(_NSLOT)]
    scratch += [pltpu.VMEM((1, _B), jnp.int32) for _ in range(_NSLOT)]
    scratch += [pltpu.VMEM((_B + 16,), jnp.float32) for _ in range(_NSLOT)]
    scratch += [pltpu.VMEM((_B, d), jnp.bfloat16) for _ in range(_NSLOT)]
    scratch += [pltpu.VMEM((_B, d), jnp.float32) for _ in range(_NF)]
    scratch += [pltpu.VMEM_SHARED((_RPC, d), jnp.float32)]
    scratch += [pltpu.SemaphoreType.DMA for _ in range(2 * _NSLOT + _NF)]

    out_type = jax.ShapeDtypeStruct((_NPAD, d), jnp.float32)
    if emit_offs:
        out_type = [out_type, jax.ShapeDtypeStruct((_NW * 16,), jnp.int32)]

    return pl.kernel(
        body,
        out_type=out_type,
        mesh=mesh,
        scratch_types=scratch,
        compiler_params=pltpu.CompilerParams(
            use_tc_tiling_on_sc=False, needs_layout_passes=False
        ),
    )


def _tc_dense(h_ref, w1t_ref, b1_ref, w2t_ref, o_ref):
    h = h_ref[...]
    z = jnp.dot(h, w1t_ref[...], preferred_element_type=jnp.float32)
    z = jnp.maximum(z + b1_ref[...], 0.0)
    o_ref[...] = jnp.dot(
        z, w2t_ref[...], preferred_element_type=jnp.float32
    ).astype(jnp.bfloat16)


@jax.jit
def kernel(x, propagation_adj, filter_vals, W1, b1, W2, b2, filter_rows, filter_cols):
    del propagation_adj
    d_hid = W1.shape[0]
    n_cls = W2.shape[0]

    spmm1 = _make_spmm(d_hid, with_bias=False, emit_offs=True)
    spmm2 = _make_spmm(n_cls, with_bias=True, emit_offs=False)

    x_b = x.astype(jnp.bfloat16)
    h1, offs = spmm1(x_b, filter_rows, filter_cols, filter_vals)

    t = pl.pallas_call(
        _tc_dense,
        out_shape=jax.ShapeDtypeStruct((_NPAD, n_cls), jnp.bfloat16),
    )(h1, W1.T, b1[None, :], W2.T)

    outp = spmm2(
        t, filter_rows, filter_cols, filter_vals, offs, b2
    )
    return outp[:_N]
